# Initial kernel scaffold; baseline (speedup 1.0000x reference)
import functools

import jax
import jax.numpy as jnp
from jax import lax
from jax.experimental import pallas as pl
from jax.experimental.pallas import tpu as pltpu
from jax.experimental.pallas import tpu_sc as plsc

B, C, H, W = 2, 128, 60, 80
HW = H * W
NVOX = 60 * 36 * 60
TPAD = HW + 16
NTILES = 32
CPT = (B * C) // NTILES
VC = 3600
NCHUNK = NVOX // VC


def _sc_body(feat, idx, out, table_v, idx_v, out_v):
    cid = lax.axis_index("c")
    sid = lax.axis_index("s")
    wid = sid * 2 + cid
    b = wid // (NTILES // B)
    cbase = (wid % (NTILES // B)) * CPT

    zeros16 = jnp.zeros((16,), jnp.float32)
    for j in range(CPT):
        pltpu.sync_copy(feat.at[b, cbase + j, :], table_v.at[j, pl.ds(0, HW)])
        table_v[j, pl.ds(HW, 16)] = zeros16

    def chunk_body(k, carry):
        vb = k * VC
        pltpu.sync_copy(idx.at[b, pl.ds(vb, VC)], idx_v)

        def gather_body(g, carry2):
            base = g * 16
            iv = idx_v[pl.ds(base, 16)]
            for j in range(CPT):
                out_v[j, pl.ds(base, 16)] = plsc.load_gather(
                    table_v.at[j], [iv])
            return carry2

        lax.fori_loop(0, VC // 16, gather_body, 0)
        for j in range(CPT):
            pltpu.sync_copy(out_v.at[j], out.at[b, cbase + j, pl.ds(vb, VC)])
        return carry

    lax.fori_loop(0, NCHUNK, chunk_body, 0)


_sc_call = pl.kernel(
    _sc_body,
    mesh=plsc.VectorSubcoreMesh(core_axis_name="c", subcore_axis_name="s"),
    out_type=jax.ShapeDtypeStruct((B, C, NVOX), jnp.float32),
    scratch_types=[
        pltpu.VMEM((CPT, TPAD), jnp.float32),
        pltpu.VMEM((VC,), jnp.int32),
        pltpu.VMEM((CPT, VC), jnp.float32),
    ],
)


@jax.jit
def kernel(feature2d, depth_mapping_3d):
    feat = feature2d.reshape(B, C, HW)
    out = _sc_call(feat, depth_mapping_3d)
    return out.reshape(B, C, 60, 36, 60)


# speedup vs baseline: 2.3691x; 2.3691x over previous
import functools

import jax
import jax.numpy as jnp
from jax import lax
from jax.experimental import pallas as pl
from jax.experimental.pallas import tpu as pltpu
from jax.experimental.pallas import tpu_sc as plsc

B, C, H, W = 2, 128, 60, 80
HW = H * W
NVOX = 60 * 36 * 60
TPAD = HW + 16
NTILES = 32
CPT = (B * C) // NTILES
VC = 3600
NCHUNK = NVOX // VC


def _sc_body(feat, idx, out, table_v, idx_v, out_v):
    cid = lax.axis_index("c")
    sid = lax.axis_index("s")
    wid = sid * 2 + cid
    b = wid // (NTILES // B)
    cbase = (wid % (NTILES // B)) * CPT

    row0 = b * C + cbase

    zeros16 = jnp.zeros((16,), jnp.float32)
    for j in range(CPT):
        pltpu.sync_copy(feat.at[pl.ds((row0 + j) * HW, HW)],
                        table_v.at[pl.ds(j * TPAD, HW)])
        table_v[pl.ds(j * TPAD + HW, 16)] = zeros16

    def chunk_body(k, carry):
        vb = k * VC
        pltpu.sync_copy(idx.at[pl.ds(b * NVOX + vb, VC)], idx_v)

        def gather_body(g, carry2):
            base = g * 16
            iv = idx_v[pl.ds(base, 16)]
            for j in range(CPT):
                out_v[pl.ds(j * VC + base, 16)] = plsc.load_gather(
                    table_v.at[pl.ds(j * TPAD, TPAD)], [iv])
            return carry2

        lax.fori_loop(0, VC // 16, gather_body, 0)
        for j in range(CPT):
            pltpu.sync_copy(out_v.at[pl.ds(j * VC, VC)],
                            out.at[pl.ds((row0 + j) * NVOX + vb, VC)])
        return carry

    lax.fori_loop(0, NCHUNK, chunk_body, 0)


_sc_call = pl.kernel(
    _sc_body,
    mesh=plsc.VectorSubcoreMesh(core_axis_name="c", subcore_axis_name="s"),
    compiler_params=pltpu.CompilerParams(needs_layout_passes=False),
    out_type=jax.ShapeDtypeStruct((B * C * NVOX,), jnp.float32),
    scratch_types=[
        pltpu.VMEM((CPT * TPAD,), jnp.float32),
        pltpu.VMEM((VC,), jnp.int32),
        pltpu.VMEM((CPT * VC,), jnp.float32),
    ],
)


@jax.jit
def kernel(feature2d, depth_mapping_3d):
    feat = feature2d.reshape(B * C * HW)
    out = _sc_call(feat, depth_mapping_3d.reshape(B * NVOX))
    return out.reshape(B, C, 60, 36, 60)


# speedup vs baseline: 2.6987x; 1.1391x over previous
import functools

import jax
import jax.numpy as jnp
from jax import lax
from jax.experimental import pallas as pl
from jax.experimental.pallas import tpu as pltpu
from jax.experimental.pallas import tpu_sc as plsc

B, C, H, W = 2, 128, 60, 80
HW = H * W
NVOX = 60 * 36 * 60
TPAD = HW + 16
NTILES = 32
CPT = (B * C) // NTILES
VC = 3600
NCHUNK = NVOX // VC


def _sc_body(feat, idx, out, table_v, idx_v, out_v,
             sem_idx0, sem_idx1, sem_out0, sem_out1):
    cid = lax.axis_index("c")
    sid = lax.axis_index("s")
    wid = sid * 2 + cid
    b = wid // (NTILES // B)
    cbase = (wid % (NTILES // B)) * CPT
    row0 = b * C + cbase

    sem_idx = (sem_idx0, sem_idx1)
    sem_out = (sem_out0, sem_out1)

    def idx_desc(k, slot):
        return pltpu.make_async_copy(
            idx.at[pl.ds(b * NVOX + k * VC, VC)],
            idx_v.at[pl.ds(slot * VC, VC)],
            sem_idx[slot])

    def out_desc(k, slot, j):
        return pltpu.make_async_copy(
            out_v.at[pl.ds((slot * CPT + j) * VC, VC)],
            out.at[pl.ds((row0 + j) * NVOX + k * VC, VC)],
            sem_out[slot])

    zeros16 = jnp.zeros((16,), jnp.float32)
    for j in range(CPT):
        pltpu.sync_copy(feat.at[pl.ds((row0 + j) * HW, HW)],
                        table_v.at[pl.ds(j * TPAD, HW)])
        table_v[pl.ds(j * TPAD + HW, 16)] = zeros16

    idx_desc(0, 0).start()

    @pl.loop(0, NCHUNK, step=2)
    def chunk_pair(k0):
        for p in range(2):
            k = k0 + p
            knext = jnp.minimum(k + 1, NCHUNK - 1)
            idx_desc(knext, 1 - p).start()
            idx_desc(k, p).wait()
            @pl.when(k0 >= 2)
            def _():
                for j in range(CPT):
                    out_desc(k, p, j).wait()

            base0 = p * CPT * VC

            @pl.loop(0, VC // 16)
            def gather_body(g):
                base = g * 16
                iv = idx_v[pl.ds(p * VC + base, 16)]
                for j in range(CPT):
                    out_v[pl.ds(base0 + j * VC + base, 16)] = (
                        plsc.load_gather(
                            table_v.at[pl.ds(j * TPAD, TPAD)], [iv]))

            for j in range(CPT):
                out_desc(k, p, j).start()

    idx_desc(NCHUNK - 1, 0).wait()
    for p in range(2):
        for j in range(CPT):
            out_desc(NCHUNK - 2 + p, p, j).wait()


_sc_call = pl.kernel(
    _sc_body,
    mesh=plsc.VectorSubcoreMesh(core_axis_name="c", subcore_axis_name="s"),
    compiler_params=pltpu.CompilerParams(needs_layout_passes=False),
    out_type=jax.ShapeDtypeStruct((B * C * NVOX,), jnp.float32),
    scratch_types=[
        pltpu.VMEM((CPT * TPAD,), jnp.float32),
        pltpu.VMEM((2 * VC,), jnp.int32),
        pltpu.VMEM((2 * CPT * VC,), jnp.float32),
        pltpu.SemaphoreType.DMA,
        pltpu.SemaphoreType.DMA,
        pltpu.SemaphoreType.DMA,
        pltpu.SemaphoreType.DMA,
    ],
)


@jax.jit
def kernel(feature2d, depth_mapping_3d):
    feat = feature2d.reshape(B * C * HW)
    out = _sc_call(feat, depth_mapping_3d.reshape(B * NVOX))
    return out.reshape(B, C, 60, 36, 60)


# speedup vs baseline: 3.9469x; 1.4626x over previous
import functools

import jax
import jax.numpy as jnp
from jax import lax
from jax.experimental import pallas as pl
from jax.experimental.pallas import tpu as pltpu
from jax.experimental.pallas import tpu_sc as plsc

B, C, H, W = 2, 128, 60, 80
HW = H * W
NVOX = 60 * 36 * 60
TPAD = HW + 16
NTILES = 32
CPT = (B * C) // NTILES
VC = 3600
NCHUNK = NVOX // VC


def _sc_body(feat, idx, out, table_v, idx_v, out_v,
             sem_idx0, sem_idx1, sem_out0, sem_out1):
    cid = lax.axis_index("c")
    sid = lax.axis_index("s")
    wid = sid * 2 + cid
    b = wid // (NTILES // B)
    cbase = (wid % (NTILES // B)) * CPT
    row0 = b * C + cbase

    sem_idx = (sem_idx0, sem_idx1)
    sem_out = (sem_out0, sem_out1)

    def idx_desc(k, slot):
        return pltpu.make_async_copy(
            idx.at[pl.ds(b * NVOX + k * VC, VC)],
            idx_v.at[pl.ds(slot * VC, VC)],
            sem_idx[slot])

    def out_desc(k, slot, j):
        return pltpu.make_async_copy(
            out_v.at[pl.ds((slot * CPT + j) * VC, VC)],
            out.at[pl.ds((row0 + j) * NVOX + k * VC, VC)],
            sem_out[slot])

    zeros16 = jnp.zeros((16,), jnp.float32)
    for j in range(CPT):
        pltpu.sync_copy(feat.at[pl.ds((row0 + j) * HW, HW)],
                        table_v.at[pl.ds(j * TPAD, HW)])
        table_v[pl.ds(j * TPAD + HW, 16)] = zeros16

    idx_desc(0, 0).start()

    @pl.loop(0, NCHUNK, step=2)
    def chunk_pair(k0):
        for p in range(2):
            k = k0 + p
            knext = jnp.minimum(k + 1, NCHUNK - 1)
            idx_desc(knext, 1 - p).start()
            idx_desc(k, p).wait()
            @pl.when(k0 >= 2)
            def _():
                for j in range(CPT):
                    out_desc(k, p, j).wait()

            base0 = p * CPT * VC

            @plsc.parallel_loop(0, VC, 16, unroll=4)
            def gather_body(base):
                iv = idx_v[pl.ds(p * VC + base, 16)]
                for j in range(CPT):
                    out_v[pl.ds(base0 + j * VC + base, 16)] = (
                        plsc.load_gather(
                            table_v.at[pl.ds(j * TPAD, TPAD)], [iv]))

            for j in range(CPT):
                out_desc(k, p, j).start()

    idx_desc(NCHUNK - 1, 0).wait()
    for p in range(2):
        for j in range(CPT):
            out_desc(NCHUNK - 2 + p, p, j).wait()


_sc_call = pl.kernel(
    _sc_body,
    mesh=plsc.VectorSubcoreMesh(core_axis_name="c", subcore_axis_name="s"),
    compiler_params=pltpu.CompilerParams(needs_layout_passes=False),
    out_type=jax.ShapeDtypeStruct((B * C * NVOX,), jnp.float32),
    scratch_types=[
        pltpu.VMEM((CPT * TPAD,), jnp.float32),
        pltpu.VMEM((2 * VC,), jnp.int32),
        pltpu.VMEM((2 * CPT * VC,), jnp.float32),
        pltpu.SemaphoreType.DMA,
        pltpu.SemaphoreType.DMA,
        pltpu.SemaphoreType.DMA,
        pltpu.SemaphoreType.DMA,
    ],
)


@jax.jit
def kernel(feature2d, depth_mapping_3d):
    feat = feature2d.reshape(B * C * HW)
    out = _sc_call(feat, depth_mapping_3d.reshape(B * NVOX))
    return out.reshape(B, C, 60, 36, 60)
